# Initial kernel scaffold; baseline (speedup 1.0000x reference)
import jax
import jax.numpy as jnp
from jax.experimental import pallas as pl


def kernel(X, edge_index, W1, b1, W2, b2):
    raise NotImplementedError("write your pallas kernel here")


# speedup vs baseline: 15.1756x; 15.1756x over previous
import functools

import jax
import jax.numpy as jnp
from jax import lax
from jax.experimental import pallas as pl
from jax.experimental.pallas import tpu as pltpu
from jax.experimental.pallas import tpu_sc as plsc

_W = 16
_BLK = 80
_NC = 2
_NS = 16
_R = 1000
_RCH = 80


def _sc_mesh():
    return plsc.VectorSubcoreMesh(core_axis_name="c", subcore_axis_name="s")


def _rows_foreach_tile(s, n, fn):
    nch = n // _RCH
    trips = (nch + _NS - 1) // _NS

    def body(i, carry):
        cid = s + i * _NS

        @pl.when(cid < nch)
        def _():
            fn(pl.multiple_of(cid * _RCH, 8))

        return carry

    lax.fori_loop(0, trips, body, 0)


@functools.lru_cache(maxsize=None)
def _make_deg(n, e):
    nw = _NC * _NS
    ept = e // nw
    nblk = ept // _BLK

    @functools.partial(
        pl.kernel,
        mesh=_sc_mesh(),
        out_type=jax.ShapeDtypeStruct((_NC, n, _W), jnp.float32),
        compiler_params=pltpu.CompilerParams(use_tc_tiling_on_sc=False),
        scratch_types=[
            pltpu.VMEM((_BLK,), jnp.int32),
            pltpu.VMEM((_BLK, _W), jnp.float32),
            pltpu.VMEM_SHARED((n, _W), jnp.float32),
        ],
    )
    def deg_kernel(dst_hbm, zeros_hbm, ones_hbm, out_hbm, didx, ones_v, acc):
        c = lax.axis_index("c")
        s = lax.axis_index("s")
        wid = c * _NS + s
        _rows_foreach_tile(s, n, lambda r0: pltpu.sync_copy(
            zeros_hbm.at[pl.ds(r0, _RCH)], acc.at[pl.ds(r0, _RCH)]))
        pltpu.sync_copy(ones_hbm, ones_v)
        plsc.subcore_barrier()
        ebase = wid * ept

        def body(i, carry):
            off = pl.multiple_of(ebase + i * _BLK, 8)
            pltpu.sync_copy(dst_hbm.at[pl.ds(off, _BLK)], didx)
            pltpu.sync_copy(ones_v, acc.at[didx], add=True)
            return carry

        lax.fori_loop(0, nblk, body, 0)
        plsc.subcore_barrier()
        _rows_foreach_tile(s, n, lambda r0: pltpu.sync_copy(
            acc.at[pl.ds(r0, _RCH)], out_hbm.at[c, pl.ds(r0, _RCH)]))

    return deg_kernel


@functools.lru_cache(maxsize=None)
def _make_prop(n, e):
    nw = _NC * _NS
    ept = e // nw
    nblk = ept // _BLK

    @functools.partial(
        pl.kernel,
        mesh=_sc_mesh(),
        out_type=jax.ShapeDtypeStruct((_NC, n, _W), jnp.float32),
        compiler_params=pltpu.CompilerParams(use_tc_tiling_on_sc=False),
        scratch_types=[
            pltpu.VMEM((_BLK,), jnp.int32),
            pltpu.VMEM((_BLK,), jnp.int32),
            pltpu.VMEM((_BLK, _W), jnp.float32),
            pltpu.VMEM_SHARED((n, _W), jnp.float32),
            pltpu.SemaphoreType.DMA,
        ],
    )
    def prop_kernel(src_hbm, dst_hbm, g_hbm, zeros_hbm, out_hbm,
                    sidx, didx, msgs, acc, sem):
        c = lax.axis_index("c")
        s = lax.axis_index("s")
        wid = c * _NS + s
        _rows_foreach_tile(s, n, lambda r0: pltpu.sync_copy(
            zeros_hbm.at[pl.ds(r0, _RCH)], acc.at[pl.ds(r0, _RCH)]))
        plsc.subcore_barrier()
        ebase = wid * ept

        def body(i, carry):
            off = pl.multiple_of(ebase + i * _BLK, 8)
            pltpu.sync_copy(src_hbm.at[pl.ds(off, _BLK)], sidx)
            pltpu.sync_copy(dst_hbm.at[pl.ds(off, _BLK)], didx)
            pltpu.async_copy(g_hbm.at[sidx], msgs, sem).wait()
            pltpu.sync_copy(msgs, acc.at[didx], add=True)
            return carry

        lax.fori_loop(0, nblk, body, 0)
        plsc.subcore_barrier()
        _rows_foreach_tile(s, n, lambda r0: pltpu.sync_copy(
            acc.at[pl.ds(r0, _RCH)], out_hbm.at[c, pl.ds(r0, _RCH)]))

    return prop_kernel


def _tc_stage1(x, w1, w2p, p0, p1):
    n, d = x.shape

    def k1(x_ref, w1_ref, w2p_ref, p0_ref, p1_ref, g0_ref, dinv_ref):
        w12 = jnp.dot(w1_ref[...], w2p_ref[...],
                      preferred_element_type=jnp.float32)
        z16 = jnp.dot(x_ref[...], w12, preferred_element_type=jnp.float32)
        deg = p0_ref[...] + p1_ref[...] + 1.0
        dinv = lax.rsqrt(deg)
        col = lax.broadcasted_iota(jnp.int32, z16.shape, 1)
        zt = z16 + jnp.where(col == 2, 1.0, 0.0)
        g0_ref[...] = dinv * zt
        dinv_ref[...] = dinv

    return pl.pallas_call(
        k1,
        grid=(n // _R,),
        in_specs=[
            pl.BlockSpec((_R, d), lambda i: (i, 0)),
            pl.BlockSpec((d, d), lambda i: (0, 0)),
            pl.BlockSpec((d, _W), lambda i: (0, 0)),
            pl.BlockSpec((_R, _W), lambda i: (i, 0)),
            pl.BlockSpec((_R, _W), lambda i: (i, 0)),
        ],
        out_specs=[
            pl.BlockSpec((_R, _W), lambda i: (i, 0)),
            pl.BlockSpec((_R, _W), lambda i: (i, 0)),
        ],
        out_shape=[
            jax.ShapeDtypeStruct((n, _W), jnp.float32),
            jax.ShapeDtypeStruct((n, _W), jnp.float32),
        ],
    )(x, w1, w2p, p0, p1)


def _tc_stage2(q0, q1, g0, dinv, b1r, w2p, b2p):
    n = q0.shape[0]
    d = b1r.shape[1]

    def k2(q0_ref, q1_ref, g0_ref, dinv_ref, b1_ref, w2p_ref, b2_ref,
           g1_ref, st_ref):
        t1 = q0_ref[...] + q1_ref[...] + g0_ref[...]
        dv = dinv_ref[...]
        g1_ref[...] = dv * dv * t1
        c16 = jnp.dot(b1_ref[...], w2p_ref[...],
                      preferred_element_type=jnp.float32)
        st_ref[...] = dv * t1[:, 2:3] * c16 + b2_ref[...]

    return pl.pallas_call(
        k2,
        grid=(n // _R,),
        in_specs=[
            pl.BlockSpec((_R, _W), lambda i: (i, 0)),
            pl.BlockSpec((_R, _W), lambda i: (i, 0)),
            pl.BlockSpec((_R, _W), lambda i: (i, 0)),
            pl.BlockSpec((_R, _W), lambda i: (i, 0)),
            pl.BlockSpec((1, d), lambda i: (0, 0)),
            pl.BlockSpec((d, _W), lambda i: (0, 0)),
            pl.BlockSpec((1, _W), lambda i: (0, 0)),
        ],
        out_specs=[
            pl.BlockSpec((_R, _W), lambda i: (i, 0)),
            pl.BlockSpec((_R, _W), lambda i: (i, 0)),
        ],
        out_shape=[
            jax.ShapeDtypeStruct((n, _W), jnp.float32),
            jax.ShapeDtypeStruct((n, _W), jnp.float32),
        ],
    )(q0, q1, g0, dinv, b1r, w2p, b2p)


def _tc_stage3(r0, r1, g1, dinv, st):
    n = r0.shape[0]

    def k3(r0_ref, r1_ref, g1_ref, dinv_ref, st_ref, out_ref):
        t2 = r0_ref[...] + r1_ref[...] + g1_ref[...]
        y = dinv_ref[...] * t2 + st_ref[...]
        a = y[:, 0:1]
        b = y[:, 1:2]
        m = jnp.maximum(a, b)
        ea = jnp.exp(a - m)
        eb = jnp.exp(b - m)
        tot = ea + eb
        col = lax.broadcasted_iota(jnp.int32, (_R, 2), 1)
        out_ref[...] = jnp.where(col == 0, ea / tot, eb / tot)

    return pl.pallas_call(
        k3,
        grid=(n // _R,),
        in_specs=[
            pl.BlockSpec((_R, _W), lambda i: (i, 0)),
            pl.BlockSpec((_R, _W), lambda i: (i, 0)),
            pl.BlockSpec((_R, _W), lambda i: (i, 0)),
            pl.BlockSpec((_R, _W), lambda i: (i, 0)),
            pl.BlockSpec((_R, _W), lambda i: (i, 0)),
        ],
        out_specs=pl.BlockSpec((_R, 2), lambda i: (i, 0)),
        out_shape=jax.ShapeDtypeStruct((n, 2), jnp.float32),
    )(r0, r1, g1, dinv, st)


def kernel(X, edge_index, W1, b1, W2, b2):
    n, _ = X.shape
    e = edge_index.shape[1]
    src = edge_index[0]
    dst = edge_index[1]

    w2p = jnp.pad(W2, ((0, 0), (0, _W - W2.shape[1])))
    b1r = b1.reshape(1, -1)
    b2p = jnp.pad(b2.reshape(1, -1), ((0, 0), (0, _W - b2.shape[0])))
    zeros_t = jnp.zeros((n, _W), jnp.float32)
    ones_t = jnp.ones((_BLK, _W), jnp.float32)

    degp = _make_deg(n, e)(dst, zeros_t, ones_t)
    g0, dinv = _tc_stage1(X, W1, w2p, degp[0], degp[1])
    t1p = _make_prop(n, e)(src, dst, g0, zeros_t)
    g1, st = _tc_stage2(t1p[0], t1p[1], g0, dinv, b1r, w2p, b2p)
    t2p = _make_prop(n, e)(src, dst, g1, zeros_t)
    return _tc_stage3(t2p[0], t2p[1], g1, dinv, st)


# speedup vs baseline: 42.1686x; 2.7787x over previous
import functools

import jax
import jax.numpy as jnp
from jax import lax
from jax.experimental import pallas as pl
from jax.experimental.pallas import tpu as pltpu
from jax.experimental.pallas import tpu_sc as plsc

_W = 16
_BLK = 125
_NBUF = 4
_NC = 2
_NS = 16
_R = 1000
_RCH = 80


def _sc_mesh():
    return plsc.VectorSubcoreMesh(core_axis_name="c", subcore_axis_name="s")


def _rows_foreach_tile(s, n, fn):
    nch = n // _RCH
    trips = (nch + _NS - 1) // _NS

    def body(i, carry):
        cid = s + i * _NS

        @pl.when(cid < nch)
        def _():
            fn(pl.multiple_of(cid * _RCH, 8))

        return carry

    lax.fori_loop(0, trips, body, 0)


@functools.lru_cache(maxsize=None)
def _make_deg(n, e):
    nw = _NC * _NS
    ept = e // nw
    nblk = ept // _BLK

    @functools.partial(
        pl.kernel,
        mesh=_sc_mesh(),
        out_type=jax.ShapeDtypeStruct((_NC, n, _W), jnp.float32),
        compiler_params=pltpu.CompilerParams(use_tc_tiling_on_sc=False),
        scratch_types=[
            pltpu.VMEM((nblk, _BLK), jnp.int32),
            pltpu.VMEM((_BLK, _W), jnp.float32),
            pltpu.VMEM_SHARED((n, _W), jnp.float32),
            [pltpu.SemaphoreType.DMA] * 2,
        ],
    )
    def deg_kernel(dst_hbm, zeros_hbm, ones_hbm, out_hbm, didx2, ones_v, acc,
                   ssem):
        c = lax.axis_index("c")
        s = lax.axis_index("s")
        wid = c * _NS + s
        _rows_foreach_tile(s, n, lambda r0: pltpu.sync_copy(
            zeros_hbm.at[pl.ds(r0, _RCH)], acc.at[pl.ds(r0, _RCH)]))
        pltpu.sync_copy(ones_hbm, ones_v)
        pltpu.sync_copy(dst_hbm.at[wid], didx2)
        plsc.subcore_barrier()

        def slot(j, b):
            pltpu.async_copy(ones_v, acc.at[didx2.at[j]], ssem[b], add=True)

            @pl.when(j >= 2)
            def _():
                pltpu.make_async_copy(
                    ones_v, acc.at[didx2.at[j - 2]], ssem[b]).wait()

        def body(k, carry):
            slot(2 * k, 0)
            slot(2 * k + 1, 1)
            return carry

        lax.fori_loop(0, nblk // 2, body, 0)
        for b in range(2):
            pltpu.make_async_copy(
                ones_v, acc.at[didx2.at[nblk - 2 + b]], ssem[b]).wait()
        plsc.subcore_barrier()
        _rows_foreach_tile(s, n, lambda r0: pltpu.sync_copy(
            acc.at[pl.ds(r0, _RCH)], out_hbm.at[c, pl.ds(r0, _RCH)]))

    return deg_kernel


@functools.lru_cache(maxsize=None)
def _make_prop(n, e):
    nw = _NC * _NS
    ept = e // nw
    nblk = ept // _BLK

    @functools.partial(
        pl.kernel,
        mesh=_sc_mesh(),
        out_type=jax.ShapeDtypeStruct((_NC, n, _W), jnp.float32),
        compiler_params=pltpu.CompilerParams(use_tc_tiling_on_sc=False),
        scratch_types=[
            pltpu.VMEM((nblk, _BLK), jnp.int32),
            pltpu.VMEM((nblk, _BLK), jnp.int32),
            [pltpu.VMEM((_BLK, _W), jnp.float32)] * _NBUF,
            pltpu.VMEM_SHARED((n, _W), jnp.float32),
            [pltpu.SemaphoreType.DMA] * _NBUF,
            [pltpu.SemaphoreType.DMA] * _NBUF,
        ],
    )
    def prop_kernel(src_hbm, dst_hbm, g_hbm, zeros_hbm, out_hbm,
                    sidx2, didx2, msgs, acc, gsem, ssem):
        c = lax.axis_index("c")
        s = lax.axis_index("s")
        wid = c * _NS + s
        _rows_foreach_tile(s, n, lambda r0: pltpu.sync_copy(
            zeros_hbm.at[pl.ds(r0, _RCH)], acc.at[pl.ds(r0, _RCH)]))
        pltpu.sync_copy(src_hbm.at[wid], sidx2)
        pltpu.sync_copy(dst_hbm.at[wid], didx2)
        plsc.subcore_barrier()

        def gather_start(j, b):
            pltpu.async_copy(g_hbm.at[sidx2.at[j]], msgs[b], gsem[b])

        def gather_wait(j, b):
            pltpu.make_async_copy(g_hbm.at[sidx2.at[j]], msgs[b], gsem[b]).wait()

        def scatter_start(j, b):
            pltpu.async_copy(msgs[b], acc.at[didx2.at[j]], ssem[b], add=True)

        def scatter_wait(j, b):
            pltpu.make_async_copy(msgs[b], acc.at[didx2.at[j]], ssem[b]).wait()

        for b in range(_NBUF - 1):
            gather_start(b, b)

        def slot(j, b):
            gather_wait(j, b)
            scatter_start(j, b)
            pb = (b - 1) % _NBUF

            @pl.when(j >= 1)
            def _():
                scatter_wait(j - 1, pb)

            @pl.when(j + _NBUF - 1 < nblk)
            def _():
                gather_start(j + _NBUF - 1, pb)

        def body(k, carry):
            for b in range(_NBUF):
                slot(k * _NBUF + b, b)
            return carry

        lax.fori_loop(0, nblk // _NBUF, body, 0)
        scatter_wait(nblk - 1, (nblk - 1) % _NBUF)
        plsc.subcore_barrier()
        _rows_foreach_tile(s, n, lambda r0: pltpu.sync_copy(
            acc.at[pl.ds(r0, _RCH)], out_hbm.at[c, pl.ds(r0, _RCH)]))

    return prop_kernel


def _tc_stage1(x, w1, w2p, p0, p1):
    n, d = x.shape

    def k1(x_ref, w1_ref, w2p_ref, p0_ref, p1_ref, g0_ref, dinv_ref):
        w12 = jnp.dot(w1_ref[...], w2p_ref[...],
                      preferred_element_type=jnp.float32)
        z16 = jnp.dot(x_ref[...], w12, preferred_element_type=jnp.float32)
        deg = p0_ref[...] + p1_ref[...] + 1.0
        dinv = lax.rsqrt(deg)
        col = lax.broadcasted_iota(jnp.int32, z16.shape, 1)
        zt = z16 + jnp.where(col == 2, 1.0, 0.0)
        g0_ref[...] = dinv * zt
        dinv_ref[...] = dinv

    return pl.pallas_call(
        k1,
        grid=(n // _R,),
        in_specs=[
            pl.BlockSpec((_R, d), lambda i: (i, 0)),
            pl.BlockSpec((d, d), lambda i: (0, 0)),
            pl.BlockSpec((d, _W), lambda i: (0, 0)),
            pl.BlockSpec((_R, _W), lambda i: (i, 0)),
            pl.BlockSpec((_R, _W), lambda i: (i, 0)),
        ],
        out_specs=[
            pl.BlockSpec((_R, _W), lambda i: (i, 0)),
            pl.BlockSpec((_R, _W), lambda i: (i, 0)),
        ],
        out_shape=[
            jax.ShapeDtypeStruct((n, _W), jnp.float32),
            jax.ShapeDtypeStruct((n, _W), jnp.float32),
        ],
    )(x, w1, w2p, p0, p1)


def _tc_stage2(q0, q1, g0, dinv, b1r, w2p, b2p):
    n = q0.shape[0]
    d = b1r.shape[1]

    def k2(q0_ref, q1_ref, g0_ref, dinv_ref, b1_ref, w2p_ref, b2_ref,
           g1_ref, st_ref):
        t1 = q0_ref[...] + q1_ref[...] + g0_ref[...]
        dv = dinv_ref[...]
        g1_ref[...] = dv * dv * t1
        c16 = jnp.dot(b1_ref[...], w2p_ref[...],
                      preferred_element_type=jnp.float32)
        st_ref[...] = dv * t1[:, 2:3] * c16 + b2_ref[...]

    return pl.pallas_call(
        k2,
        grid=(n // _R,),
        in_specs=[
            pl.BlockSpec((_R, _W), lambda i: (i, 0)),
            pl.BlockSpec((_R, _W), lambda i: (i, 0)),
            pl.BlockSpec((_R, _W), lambda i: (i, 0)),
            pl.BlockSpec((_R, _W), lambda i: (i, 0)),
            pl.BlockSpec((1, d), lambda i: (0, 0)),
            pl.BlockSpec((d, _W), lambda i: (0, 0)),
            pl.BlockSpec((1, _W), lambda i: (0, 0)),
        ],
        out_specs=[
            pl.BlockSpec((_R, _W), lambda i: (i, 0)),
            pl.BlockSpec((_R, _W), lambda i: (i, 0)),
        ],
        out_shape=[
            jax.ShapeDtypeStruct((n, _W), jnp.float32),
            jax.ShapeDtypeStruct((n, _W), jnp.float32),
        ],
    )(q0, q1, g0, dinv, b1r, w2p, b2p)


def _tc_stage3(r0, r1, g1, dinv, st):
    n = r0.shape[0]

    def k3(r0_ref, r1_ref, g1_ref, dinv_ref, st_ref, out_ref):
        t2 = r0_ref[...] + r1_ref[...] + g1_ref[...]
        y = dinv_ref[...] * t2 + st_ref[...]
        a = y[:, 0:1]
        b = y[:, 1:2]
        m = jnp.maximum(a, b)
        ea = jnp.exp(a - m)
        eb = jnp.exp(b - m)
        tot = ea + eb
        col = lax.broadcasted_iota(jnp.int32, (_R, 2), 1)
        out_ref[...] = jnp.where(col == 0, ea / tot, eb / tot)

    return pl.pallas_call(
        k3,
        grid=(n // _R,),
        in_specs=[
            pl.BlockSpec((_R, _W), lambda i: (i, 0)),
            pl.BlockSpec((_R, _W), lambda i: (i, 0)),
            pl.BlockSpec((_R, _W), lambda i: (i, 0)),
            pl.BlockSpec((_R, _W), lambda i: (i, 0)),
            pl.BlockSpec((_R, _W), lambda i: (i, 0)),
        ],
        out_specs=pl.BlockSpec((_R, 2), lambda i: (i, 0)),
        out_shape=jax.ShapeDtypeStruct((n, 2), jnp.float32),
    )(r0, r1, g1, dinv, st)


def kernel(X, edge_index, W1, b1, W2, b2):
    n, _ = X.shape
    e = edge_index.shape[1]
    nw = _NC * _NS
    nblk = e // nw // _BLK
    src = edge_index[0].reshape(nw, nblk, _BLK)
    dst = edge_index[1].reshape(nw, nblk, _BLK)

    w2p = jnp.pad(W2, ((0, 0), (0, _W - W2.shape[1])))
    b1r = b1.reshape(1, -1)
    b2p = jnp.pad(b2.reshape(1, -1), ((0, 0), (0, _W - b2.shape[0])))
    zeros_t = jnp.zeros((n, _W), jnp.float32)
    ones_t = jnp.ones((_BLK, _W), jnp.float32)

    degp = _make_deg(n, e)(dst, zeros_t, ones_t)
    g0, dinv = _tc_stage1(X, W1, w2p, degp[0], degp[1])
    t1p = _make_prop(n, e)(src, dst, g0, zeros_t)
    g1, st = _tc_stage2(t1p[0], t1p[1], g0, dinv, b1r, w2p, b2p)
    t2p = _make_prop(n, e)(src, dst, g1, zeros_t)
    return _tc_stage3(t2p[0], t2p[1], g1, dinv, st)
